# baseline (speedup 1.0000x reference)
import functools

import jax
import jax.numpy as jnp
from jax import lax
from jax.experimental import pallas as pl
from jax.experimental.pallas import tpu as pltpu
from jax.experimental.pallas import tpu_sc as plsc

_D = 64
_DP = 128
_NC = 2
_NS = 16
_NW = _NC * _NS
_B_TOTAL = 16384 * 50
_B_PER_W = _B_TOTAL // _NW
_CHUNK = 400
_N_CHUNKS = _B_PER_W // _CHUNK

_mesh = plsc.VectorSubcoreMesh(core_axis_name="c", subcore_axis_name="s")


@functools.partial(
    pl.kernel,
    mesh=_mesh,
    out_type=jax.ShapeDtypeStruct((_B_TOTAL, _DP), jnp.float32),
    scratch_types=[
        pltpu.VMEM((_CHUNK,), jnp.int32),
        pltpu.VMEM((2, _CHUNK, _DP), jnp.float32),
        pltpu.SemaphoreType.DMA,
        pltpu.SemaphoreType.DMA,
        pltpu.SemaphoreType.DMA,
    ],
)
def _embed_gather(table_hbm, idx_hbm, out_hbm, idx_v, rows_v, sem_g, sem_s0,
                  sem_s1):
    wid = lax.axis_index("s") * _NC + lax.axis_index("c")
    base = wid * _B_PER_W
    sem_s = (sem_s0, sem_s1)

    def step(j, carry):
        for b in range(2):
            g = 2 * j + b
            off = base + g * _CHUNK
            pltpu.sync_copy(idx_hbm.at[pl.ds(off, _CHUNK)], idx_v)

            @pl.when(j > 0)
            def _drain():
                pltpu.make_async_copy(
                    rows_v.at[b],
                    out_hbm.at[pl.ds(off - 2 * _CHUNK, _CHUNK)],
                    sem_s[b]).wait()

            pltpu.async_copy(table_hbm.at[idx_v], rows_v.at[b], sem_g).wait()
            pltpu.async_copy(rows_v.at[b], out_hbm.at[pl.ds(off, _CHUNK)],
                             sem_s[b])
        return carry

    lax.fori_loop(0, _N_CHUNKS // 2, step, 0)
    for b in range(2):
        g = _N_CHUNKS - 2 + b
        pltpu.make_async_copy(
            rows_v.at[b], out_hbm.at[pl.ds(base + g * _CHUNK, _CHUNK)],
            sem_s[b]).wait()


def kernel(table, indices):
    table_p = jnp.pad(table, ((0, 0), (0, _DP - _D)))
    idx_flat = indices.reshape(-1).astype(jnp.int32)
    out = _embed_gather(table_p, idx_flat)
    return out[:, :_D].reshape(*indices.shape, _D)


# speedup vs baseline: 1.1274x; 1.1274x over previous
import functools

import jax
import jax.numpy as jnp
from jax import lax
from jax.experimental import pallas as pl
from jax.experimental.pallas import tpu as pltpu
from jax.experimental.pallas import tpu_sc as plsc

_D = 64
_NC = 2
_NS = 16
_NW = _NC * _NS
_B_TOTAL = 16384 * 50
_B_PER_W = _B_TOTAL // _NW
_CHUNK = 512
_N_CHUNKS = _B_PER_W // _CHUNK

_mesh = plsc.VectorSubcoreMesh(core_axis_name="c", subcore_axis_name="s")


@functools.partial(
    pl.kernel,
    mesh=_mesh,
    out_type=jax.ShapeDtypeStruct((_B_TOTAL, _D), jnp.float32),
    scratch_types=[
        pltpu.VMEM((_CHUNK,), jnp.int32),
        pltpu.VMEM((2, _CHUNK, _D), jnp.float32),
        pltpu.SemaphoreType.DMA,
        pltpu.SemaphoreType.DMA,
        pltpu.SemaphoreType.DMA,
    ],
    compiler_params=pltpu.CompilerParams(use_tc_tiling_on_sc=False),
)
def _embed_gather(table_hbm, idx_hbm, out_hbm, idx_v, rows_v, sem_g, sem_s0,
                  sem_s1):
    wid = lax.axis_index("s") * _NC + lax.axis_index("c")
    base = wid * _B_PER_W
    sem_s = (sem_s0, sem_s1)

    def step(j, carry):
        for b in range(2):
            g = 2 * j + b
            off = base + g * _CHUNK
            pltpu.sync_copy(idx_hbm.at[pl.ds(off, _CHUNK)], idx_v)

            @pl.when(j > 0)
            def _drain():
                pltpu.make_async_copy(
                    rows_v.at[b],
                    out_hbm.at[pl.ds(off - 2 * _CHUNK, _CHUNK)],
                    sem_s[b]).wait()

            pltpu.async_copy(table_hbm.at[idx_v], rows_v.at[b], sem_g).wait()
            pltpu.async_copy(rows_v.at[b], out_hbm.at[pl.ds(off, _CHUNK)],
                             sem_s[b])
        return carry

    lax.fori_loop(0, _N_CHUNKS // 2, step, 0)
    for b in range(2):
        g = _N_CHUNKS - 2 + b
        pltpu.make_async_copy(
            rows_v.at[b], out_hbm.at[pl.ds(base + g * _CHUNK, _CHUNK)],
            sem_s[b]).wait()


def kernel(table, indices):
    idx_flat = indices.reshape(-1).astype(jnp.int32)
    out = _embed_gather(table, idx_flat)
    return out.reshape(*indices.shape, _D)


# speedup vs baseline: 1.1628x; 1.0314x over previous
import functools

import jax
import jax.numpy as jnp
from jax import lax
from jax.experimental import pallas as pl
from jax.experimental.pallas import tpu as pltpu
from jax.experimental.pallas import tpu_sc as plsc

_D = 64
_DP = 128
_H = 50
_B = 16384
_NC = 2
_NS = 16
_NW = _NC * _NS
_BW = _B // _NW
_BC = 128

_mesh = plsc.VectorSubcoreMesh(core_axis_name="c", subcore_axis_name="s")


@functools.partial(
    pl.kernel,
    mesh=_mesh,
    out_type=jax.ShapeDtypeStruct((_H, _D, _B), jnp.float32),
    scratch_types=[
        pltpu.VMEM((4, _H, _BC), jnp.int32),
        pltpu.VMEM((2, _BC, _DP), jnp.float32),
        pltpu.VMEM((2, _D, _BC), jnp.float32),
        pltpu.SemaphoreType.DMA,
        pltpu.SemaphoreType.DMA,
        pltpu.SemaphoreType.DMA,
        pltpu.SemaphoreType.DMA,
    ],
    compiler_params=pltpu.CompilerParams(needs_layout_passes=False),
)
def _embed_gather(table_hbm, idx_hbm, out_hbm, idx_v, rows_v, plane_v,
                  sem_g0, sem_g1, sem_w0, sem_w1):
    wid = lax.axis_index("s") * _NC + lax.axis_index("c")
    b0 = wid * _BW
    sem_g = (sem_g0, sem_g1)
    sem_w = (sem_w0, sem_w1)
    iota = lax.iota(jnp.int32, 16)
    rowk = [iota + 16 * k for k in range(4)]

    for c in range(4):
        pltpu.sync_copy(idx_hbm.at[:, pl.ds(b0 + c * _BC, _BC)], idx_v.at[c])

    def gather(h, c, p):
        return pltpu.make_async_copy(
            table_hbm.at[idx_v.at[c, h]], rows_v.at[p], sem_g[p])

    gather(0, 0, 0).start()

    def step(j, carry):
        for c in range(4):
            p = c % 2
            gather(j, c, p).wait()
            if c < 3:
                gather(j, c + 1, 1 - p).start()
            else:
                @pl.when(j + 1 < _H)
                def _pre():
                    gather(j + 1, 0, 1 - p).start()

            jd, cd = (j, c - 2) if c >= 2 else (j - 1, c + 2)

            def _drain():
                pltpu.make_async_copy(
                    plane_v.at[p],
                    out_hbm.at[jd, :, pl.ds(b0 + cd * _BC, _BC)],
                    sem_w[p]).wait()

            if c >= 2:
                _drain()
            else:
                pl.when(j > 0)(_drain)

            @plsc.parallel_loop(0, _BC, unroll=8)
            def tb(b):
                col = jnp.zeros((16,), jnp.int32) + b
                for k in range(4):
                    v = rows_v[p, b, pl.ds(16 * k, 16)]
                    plsc.store_scatter(plane_v.at[p], [rowk[k], col], v)
            pltpu.async_copy(plane_v.at[p],
                             out_hbm.at[j, :, pl.ds(b0 + c * _BC, _BC)],
                             sem_w[p])
        return carry

    lax.fori_loop(0, _H, step, 0)
    for c in (2, 3):
        pltpu.make_async_copy(
            plane_v.at[c % 2],
            out_hbm.at[_H - 1, :, pl.ds(b0 + c * _BC, _BC)],
            sem_w[c % 2]).wait()


def kernel(table, indices):
    table_p = jnp.pad(table, ((0, 0), (0, _DP - _D)))
    idx_t = indices.T.astype(jnp.int32)
    out_phys = _embed_gather(table_p, idx_t)
    return jnp.transpose(out_phys, (2, 0, 1))
